# Initial kernel scaffold; baseline (speedup 1.0000x reference)
import jax
import jax.numpy as jnp
from jax.experimental import pallas as pl


def kernel(x, edge_index, W1, b1, W2, b2):
    raise NotImplementedError("write your pallas kernel here")


# speedup vs baseline: 21.7952x; 21.7952x over previous
import functools

import jax
import jax.numpy as jnp
from jax import lax
from jax.experimental import pallas as pl
from jax.experimental.pallas import tpu as pltpu
from jax.experimental.pallas import tpu_sc as plsc

N_NODES = 10000
N_EDGES = 320000
D_IN = 128
D_HID = 128
D_OUT = 64

NC = 2
NS = 16
NW = NC * NS
E_PER_W = N_EDGES // NW
K = 80
NCH = E_PER_W // K
R_BIG = 640
R_LAST = N_NODES - 15 * R_BIG
DEG_D = 16
BM = 1000


def _make_agg(D):
  mesh = plsc.VectorSubcoreMesh(core_axis_name="c", subcore_axis_name="s")

  @functools.partial(
      pl.kernel, mesh=mesh,
      out_type=jax.ShapeDtypeStruct((2 * N_NODES, D), jnp.float32),
      compiler_params=pltpu.CompilerParams(use_tc_tiling_on_sc=False),
      scratch_types=[
          pltpu.VMEM((NCH, K), jnp.int32),
          pltpu.VMEM((NCH, K), jnp.int32),
          pltpu.VMEM((K, D), jnp.float32),
          pltpu.VMEM_SHARED((N_NODES, D), jnp.float32),
          pltpu.SemaphoreType.DMA,
      ])
  def agg(h_hbm, src_hbm, dst_hbm, z_hbm, out_hbm, src_v, dst_v, rows_v, acc,
          sem):
    c = lax.axis_index("c")
    s = lax.axis_index("s")
    wid = s * NC + c
    rbase = s * R_BIG
    @pl.when(s < 15)
    def _():
      pltpu.sync_copy(z_hbm.at[pl.ds(rbase, R_BIG)],
                      acc.at[pl.ds(rbase, R_BIG)])

    @pl.when(s == 15)
    def _():
      pltpu.sync_copy(z_hbm.at[pl.ds(rbase, R_LAST)],
                      acc.at[pl.ds(rbase, R_LAST)])

    pltpu.sync_copy(src_hbm.at[wid], src_v)
    pltpu.sync_copy(dst_hbm.at[wid], dst_v)
    plsc.subcore_barrier()

    def body(j, carry):
      pltpu.async_copy(h_hbm.at[src_v.at[j]], rows_v, sem).wait()
      pltpu.sync_copy(rows_v, acc.at[dst_v.at[j]], add=True)
      return carry

    lax.fori_loop(0, NCH, body, 0)
    plsc.subcore_barrier()

    @pl.when(s < 15)
    def _():
      pltpu.sync_copy(acc.at[pl.ds(rbase, R_BIG)],
                      out_hbm.at[pl.ds(c * N_NODES + rbase, R_BIG)])

    @pl.when(s == 15)
    def _():
      pltpu.sync_copy(acc.at[pl.ds(rbase, R_LAST)],
                      out_hbm.at[pl.ds(c * N_NODES + rbase, R_LAST)])

  return agg


_agg128 = _make_agg(D_HID)
_agg64 = _make_agg(D_OUT)


def _make_deg():
  mesh = plsc.VectorSubcoreMesh(core_axis_name="c", subcore_axis_name="s")

  @functools.partial(
      pl.kernel, mesh=mesh,
      out_type=jax.ShapeDtypeStruct((2 * N_NODES, DEG_D), jnp.float32),
      compiler_params=pltpu.CompilerParams(use_tc_tiling_on_sc=False),
      scratch_types=[
          pltpu.VMEM((NCH, K), jnp.int32),
          pltpu.VMEM((K, DEG_D), jnp.float32),
          pltpu.VMEM_SHARED((N_NODES, DEG_D), jnp.float32),
      ])
  def deg(dst_hbm, ones_hbm, z_hbm, out_hbm, dst_v, ones_v, acc):
    c = lax.axis_index("c")
    s = lax.axis_index("s")
    wid = s * NC + c
    rbase = s * R_BIG

    @pl.when(s < 15)
    def _():
      pltpu.sync_copy(z_hbm.at[pl.ds(rbase, R_BIG)],
                      acc.at[pl.ds(rbase, R_BIG)])

    @pl.when(s == 15)
    def _():
      pltpu.sync_copy(z_hbm.at[pl.ds(rbase, R_LAST)],
                      acc.at[pl.ds(rbase, R_LAST)])

    pltpu.sync_copy(dst_hbm.at[wid], dst_v)
    pltpu.sync_copy(ones_hbm, ones_v)
    plsc.subcore_barrier()

    def body(j, carry):
      pltpu.sync_copy(ones_v, acc.at[dst_v.at[j]], add=True)
      return carry

    lax.fori_loop(0, NCH, body, 0)
    plsc.subcore_barrier()

    @pl.when(s < 15)
    def _():
      pltpu.sync_copy(acc.at[pl.ds(rbase, R_BIG)],
                      out_hbm.at[pl.ds(c * N_NODES + rbase, R_BIG)])

    @pl.when(s == 15)
    def _():
      pltpu.sync_copy(acc.at[pl.ds(rbase, R_LAST)],
                      out_hbm.at[pl.ds(c * N_NODES + rbase, R_LAST)])

  return deg


_deg = _make_deg()


def _dinv_block(deg_ref):
  d = deg_ref[0, :, 0] + deg_ref[1, :, 0] + 1.0
  return lax.rsqrt(d)[:, None]


def _mm1_body(x_ref, w_ref, deg_ref, o_ref):
  h = jnp.dot(x_ref[...], w_ref[...], preferred_element_type=jnp.float32)
  o_ref[...] = h * _dinv_block(deg_ref)


def _mid_body(p_ref, h_ref, deg_ref, b_ref, w_ref, o_ref):
  dinv = _dinv_block(deg_ref)
  agg = p_ref[0] + p_ref[1] + h_ref[...]
  z = jnp.maximum(agg * dinv + b_ref[...], 0.0)
  o_ref[...] = jnp.dot(z * dinv, w_ref[...], preferred_element_type=jnp.float32)


def _fin_body(q_ref, h_ref, deg_ref, b_ref, o_ref):
  dinv = _dinv_block(deg_ref)
  o_ref[...] = (q_ref[0] + q_ref[1] + h_ref[...]) * dinv + b_ref[...]


def _mm1(x, W1, deg_raw):
  return pl.pallas_call(
      _mm1_body,
      grid=(N_NODES // BM,),
      in_specs=[
          pl.BlockSpec((BM, D_IN), lambda i: (i, 0)),
          pl.BlockSpec((D_IN, D_HID), lambda i: (0, 0)),
          pl.BlockSpec((2, BM, DEG_D), lambda i: (0, i, 0)),
      ],
      out_specs=pl.BlockSpec((BM, D_HID), lambda i: (i, 0)),
      out_shape=jax.ShapeDtypeStruct((N_NODES, D_HID), jnp.float32),
  )(x, W1, deg_raw)


def _mid(p1, h1s, deg_raw, b1, W2):
  return pl.pallas_call(
      _mid_body,
      grid=(N_NODES // BM,),
      in_specs=[
          pl.BlockSpec((2, BM, D_HID), lambda i: (0, i, 0)),
          pl.BlockSpec((BM, D_HID), lambda i: (i, 0)),
          pl.BlockSpec((2, BM, DEG_D), lambda i: (0, i, 0)),
          pl.BlockSpec((1, D_HID), lambda i: (0, 0)),
          pl.BlockSpec((D_HID, D_OUT), lambda i: (0, 0)),
      ],
      out_specs=pl.BlockSpec((BM, D_OUT), lambda i: (i, 0)),
      out_shape=jax.ShapeDtypeStruct((N_NODES, D_OUT), jnp.float32),
  )(p1, h1s, deg_raw, b1, W2)


def _fin(q, h2s, deg_raw, b2):
  return pl.pallas_call(
      _fin_body,
      grid=(N_NODES // BM,),
      in_specs=[
          pl.BlockSpec((2, BM, D_OUT), lambda i: (0, i, 0)),
          pl.BlockSpec((BM, D_OUT), lambda i: (i, 0)),
          pl.BlockSpec((2, BM, DEG_D), lambda i: (0, i, 0)),
          pl.BlockSpec((1, D_OUT), lambda i: (0, 0)),
      ],
      out_specs=pl.BlockSpec((BM, D_OUT), lambda i: (i, 0)),
      out_shape=jax.ShapeDtypeStruct((N_NODES, D_OUT), jnp.float32),
  )(q, h2s, deg_raw, b2)


def kernel(x, edge_index, W1, b1, W2, b2):
  src = edge_index[0].reshape(NW, NCH, K)
  dst = edge_index[1].reshape(NW, NCH, K)
  ones16 = jnp.ones((K, DEG_D), jnp.float32)
  z16 = jnp.zeros((N_NODES, DEG_D), jnp.float32)
  z128 = jnp.zeros((N_NODES, D_HID), jnp.float32)
  z64 = jnp.zeros((N_NODES, D_OUT), jnp.float32)

  deg_raw = _deg(dst, ones16, z16).reshape(2, N_NODES, DEG_D)
  h1s = _mm1(x, W1, deg_raw)
  p1 = _agg128(h1s, src, dst, z128).reshape(2, N_NODES, D_HID)
  h2s = _mid(p1, h1s, deg_raw, b1.reshape(1, D_HID), W2)
  p2 = _agg64(h2s, src, dst, z64).reshape(2, N_NODES, D_OUT)
  return _fin(p2, h2s, deg_raw, b2.reshape(1, D_OUT))


# speedup vs baseline: 26.3105x; 1.2072x over previous
import functools

import jax
import jax.numpy as jnp
from jax import lax
from jax.experimental import pallas as pl
from jax.experimental.pallas import tpu as pltpu
from jax.experimental.pallas import tpu_sc as plsc

N_NODES = 10000
N_EDGES = 320000
D_IN = 128
D_HID = 128
D_OUT = 64

NC = 2
NS = 16
NW = NC * NS
E_PER_W = N_EDGES // NW
K = 80
NCH = E_PER_W // K
R_BIG = 640
R_LAST = N_NODES - 15 * R_BIG
DEG_D = 16
BM = 1000


def _make_agg(D):
  mesh = plsc.VectorSubcoreMesh(core_axis_name="c", subcore_axis_name="s")

  @functools.partial(
      pl.kernel, mesh=mesh,
      out_type=jax.ShapeDtypeStruct((2 * N_NODES, D), jnp.float32),
      compiler_params=pltpu.CompilerParams(use_tc_tiling_on_sc=False),
      scratch_types=[
          pltpu.VMEM((NCH, K), jnp.int32),
          pltpu.VMEM((NCH, K), jnp.int32),
          pltpu.VMEM((K, D), jnp.float32),
          pltpu.VMEM((K, D), jnp.float32),
          pltpu.VMEM_SHARED((N_NODES, D), jnp.float32),
          pltpu.SemaphoreType.DMA,
          pltpu.SemaphoreType.DMA,
      ])
  def agg(h_hbm, src_hbm, dst_hbm, z_hbm, out_hbm, src_v, dst_v, rows0, rows1,
          acc, sem0, sem1):
    c = lax.axis_index("c")
    s = lax.axis_index("s")
    wid = s * NC + c
    rbase = s * R_BIG
    @pl.when(s < 15)
    def _():
      pltpu.sync_copy(z_hbm.at[pl.ds(rbase, R_BIG)],
                      acc.at[pl.ds(rbase, R_BIG)])

    @pl.when(s == 15)
    def _():
      pltpu.sync_copy(z_hbm.at[pl.ds(rbase, R_LAST)],
                      acc.at[pl.ds(rbase, R_LAST)])

    pltpu.sync_copy(src_hbm.at[wid], src_v)
    pltpu.sync_copy(dst_hbm.at[wid], dst_v)
    plsc.subcore_barrier()

    rows = (rows0, rows1)
    sems = (sem0, sem1)

    def start_gather(j, b):
      pltpu.async_copy(h_hbm.at[src_v.at[j]], rows[b], sems[b])

    def wait_gather(b):
      pltpu.make_async_copy(h_hbm.at[src_v.at[0]], rows[b], sems[b]).wait()

    def scatter(j, b):
      pltpu.sync_copy(rows[b], acc.at[dst_v.at[j]], add=True)

    start_gather(0, 0)

    def body(i, carry):
      j = 2 * i
      wait_gather(0)
      start_gather(j + 1, 1)
      scatter(j, 0)
      wait_gather(1)
      start_gather(j + 2, 0)
      scatter(j + 1, 1)
      return carry

    lax.fori_loop(0, (NCH - 1) // 2, body, 0)
    wait_gather(0)
    scatter(NCH - 1, 0)
    plsc.subcore_barrier()

    @pl.when(s < 15)
    def _():
      pltpu.sync_copy(acc.at[pl.ds(rbase, R_BIG)],
                      out_hbm.at[pl.ds(c * N_NODES + rbase, R_BIG)])

    @pl.when(s == 15)
    def _():
      pltpu.sync_copy(acc.at[pl.ds(rbase, R_LAST)],
                      out_hbm.at[pl.ds(c * N_NODES + rbase, R_LAST)])

  return agg


_agg128 = _make_agg(D_HID)
_agg64 = _make_agg(D_OUT)


def _make_deg():
  mesh = plsc.VectorSubcoreMesh(core_axis_name="c", subcore_axis_name="s")

  @functools.partial(
      pl.kernel, mesh=mesh,
      out_type=jax.ShapeDtypeStruct((2 * N_NODES, DEG_D), jnp.float32),
      compiler_params=pltpu.CompilerParams(use_tc_tiling_on_sc=False),
      scratch_types=[
          pltpu.VMEM((NCH, K), jnp.int32),
          pltpu.VMEM((K, DEG_D), jnp.float32),
          pltpu.VMEM_SHARED((N_NODES, DEG_D), jnp.float32),
      ])
  def deg(dst_hbm, ones_hbm, z_hbm, out_hbm, dst_v, ones_v, acc):
    c = lax.axis_index("c")
    s = lax.axis_index("s")
    wid = s * NC + c
    rbase = s * R_BIG

    @pl.when(s < 15)
    def _():
      pltpu.sync_copy(z_hbm.at[pl.ds(rbase, R_BIG)],
                      acc.at[pl.ds(rbase, R_BIG)])

    @pl.when(s == 15)
    def _():
      pltpu.sync_copy(z_hbm.at[pl.ds(rbase, R_LAST)],
                      acc.at[pl.ds(rbase, R_LAST)])

    pltpu.sync_copy(dst_hbm.at[wid], dst_v)
    pltpu.sync_copy(ones_hbm, ones_v)
    plsc.subcore_barrier()

    def body(j, carry):
      pltpu.sync_copy(ones_v, acc.at[dst_v.at[j]], add=True)
      return carry

    lax.fori_loop(0, NCH, body, 0)
    plsc.subcore_barrier()

    @pl.when(s < 15)
    def _():
      pltpu.sync_copy(acc.at[pl.ds(rbase, R_BIG)],
                      out_hbm.at[pl.ds(c * N_NODES + rbase, R_BIG)])

    @pl.when(s == 15)
    def _():
      pltpu.sync_copy(acc.at[pl.ds(rbase, R_LAST)],
                      out_hbm.at[pl.ds(c * N_NODES + rbase, R_LAST)])

  return deg


_deg = _make_deg()


def _dinv_block(deg_ref):
  d = deg_ref[0, :, 0] + deg_ref[1, :, 0] + 1.0
  return lax.rsqrt(d)[:, None]


def _mm1_body(x_ref, w_ref, deg_ref, o_ref):
  h = jnp.dot(x_ref[...], w_ref[...], preferred_element_type=jnp.float32)
  o_ref[...] = h * _dinv_block(deg_ref)


def _mid_body(p_ref, h_ref, deg_ref, b_ref, w_ref, o_ref):
  dinv = _dinv_block(deg_ref)
  agg = p_ref[0] + p_ref[1] + h_ref[...]
  z = jnp.maximum(agg * dinv + b_ref[...], 0.0)
  o_ref[...] = jnp.dot(z * dinv, w_ref[...], preferred_element_type=jnp.float32)


def _fin_body(q_ref, h_ref, deg_ref, b_ref, o_ref):
  dinv = _dinv_block(deg_ref)
  o_ref[...] = (q_ref[0] + q_ref[1] + h_ref[...]) * dinv + b_ref[...]


def _mm1(x, W1, deg_raw):
  return pl.pallas_call(
      _mm1_body,
      grid=(N_NODES // BM,),
      in_specs=[
          pl.BlockSpec((BM, D_IN), lambda i: (i, 0)),
          pl.BlockSpec((D_IN, D_HID), lambda i: (0, 0)),
          pl.BlockSpec((2, BM, DEG_D), lambda i: (0, i, 0)),
      ],
      out_specs=pl.BlockSpec((BM, D_HID), lambda i: (i, 0)),
      out_shape=jax.ShapeDtypeStruct((N_NODES, D_HID), jnp.float32),
  )(x, W1, deg_raw)


def _mid(p1, h1s, deg_raw, b1, W2):
  return pl.pallas_call(
      _mid_body,
      grid=(N_NODES // BM,),
      in_specs=[
          pl.BlockSpec((2, BM, D_HID), lambda i: (0, i, 0)),
          pl.BlockSpec((BM, D_HID), lambda i: (i, 0)),
          pl.BlockSpec((2, BM, DEG_D), lambda i: (0, i, 0)),
          pl.BlockSpec((1, D_HID), lambda i: (0, 0)),
          pl.BlockSpec((D_HID, D_OUT), lambda i: (0, 0)),
      ],
      out_specs=pl.BlockSpec((BM, D_OUT), lambda i: (i, 0)),
      out_shape=jax.ShapeDtypeStruct((N_NODES, D_OUT), jnp.float32),
  )(p1, h1s, deg_raw, b1, W2)


def _fin(q, h2s, deg_raw, b2):
  return pl.pallas_call(
      _fin_body,
      grid=(N_NODES // BM,),
      in_specs=[
          pl.BlockSpec((2, BM, D_OUT), lambda i: (0, i, 0)),
          pl.BlockSpec((BM, D_OUT), lambda i: (i, 0)),
          pl.BlockSpec((2, BM, DEG_D), lambda i: (0, i, 0)),
          pl.BlockSpec((1, D_OUT), lambda i: (0, 0)),
      ],
      out_specs=pl.BlockSpec((BM, D_OUT), lambda i: (i, 0)),
      out_shape=jax.ShapeDtypeStruct((N_NODES, D_OUT), jnp.float32),
  )(q, h2s, deg_raw, b2)


def kernel(x, edge_index, W1, b1, W2, b2):
  src = edge_index[0].reshape(NW, NCH, K)
  dst = edge_index[1].reshape(NW, NCH, K)
  ones16 = jnp.ones((K, DEG_D), jnp.float32)
  z16 = jnp.zeros((N_NODES, DEG_D), jnp.float32)
  z128 = jnp.zeros((N_NODES, D_HID), jnp.float32)
  z64 = jnp.zeros((N_NODES, D_OUT), jnp.float32)

  deg_raw = _deg(dst, ones16, z16).reshape(2, N_NODES, DEG_D)
  h1s = _mm1(x, W1, deg_raw)
  p1 = _agg128(h1s, src, dst, z128).reshape(2, N_NODES, D_HID)
  h2s = _mid(p1, h1s, deg_raw, b1.reshape(1, D_HID), W2)
  p2 = _agg64(h2s, src, dst, z64).reshape(2, N_NODES, D_OUT)
  return _fin(p2, h2s, deg_raw, b2.reshape(1, D_OUT))


# speedup vs baseline: 28.2610x; 1.0741x over previous
import functools

import jax
import jax.numpy as jnp
from jax import lax
from jax.experimental import pallas as pl
from jax.experimental.pallas import tpu as pltpu
from jax.experimental.pallas import tpu_sc as plsc

N_NODES = 10000
N_EDGES = 320000
D_IN = 128
D_HID = 128
D_OUT = 64

NC = 2
NS = 16
NW = NC * NS
E_PER_W = N_EDGES // NW
K = 80
NCH = E_PER_W // K
R_BIG = 640
R_LAST = N_NODES - 15 * R_BIG
DEG_D = 16
NBUF = 2
NBUF_DEG = 4
BM = 1000


def _make_agg(D):
  mesh = plsc.VectorSubcoreMesh(core_axis_name="c", subcore_axis_name="s")

  @functools.partial(
      pl.kernel, mesh=mesh,
      out_type=jax.ShapeDtypeStruct((2 * N_NODES, D), jnp.float32),
      compiler_params=pltpu.CompilerParams(use_tc_tiling_on_sc=False),
      scratch_types=[
          pltpu.VMEM((NCH, K), jnp.int32),
          pltpu.VMEM((NCH, K), jnp.int32),
          [pltpu.VMEM((K, D), jnp.float32) for _ in range(NBUF)],
          [pltpu.SemaphoreType.DMA for _ in range(NBUF)],
          [pltpu.SemaphoreType.DMA for _ in range(NBUF)],
          pltpu.VMEM_SHARED((N_NODES, D), jnp.float32),
      ])
  def agg(h_hbm, src_hbm, dst_hbm, z_hbm, out_hbm, src_v, dst_v, rows, gsems,
          ssems, acc):
    c = lax.axis_index("c")
    s = lax.axis_index("s")
    wid = s * NC + c
    rbase = s * R_BIG
    @pl.when(s < 15)
    def _():
      pltpu.sync_copy(z_hbm.at[pl.ds(rbase, R_BIG)],
                      acc.at[pl.ds(rbase, R_BIG)])

    @pl.when(s == 15)
    def _():
      pltpu.sync_copy(z_hbm.at[pl.ds(rbase, R_LAST)],
                      acc.at[pl.ds(rbase, R_LAST)])

    pltpu.sync_copy(src_hbm.at[wid], src_v)
    pltpu.sync_copy(dst_hbm.at[wid], dst_v)
    plsc.subcore_barrier()

    def start_gather(j, b):
      pltpu.async_copy(h_hbm.at[src_v.at[j]], rows[b], gsems[b])

    def wait_gather(b):
      pltpu.make_async_copy(h_hbm.at[src_v.at[0]], rows[b], gsems[b]).wait()

    def start_scatter(j, b):
      pltpu.async_copy(rows[b], acc.at[dst_v.at[j]], ssems[b], add=True)

    def wait_scatter(j, b):
      pltpu.make_async_copy(rows[b], acc.at[dst_v.at[j]], ssems[b]).wait()

    for q in range(NBUF):
      start_gather(q, q)

    def body(i, carry):
      j = NBUF * i
      for q in range(NBUF):
        wait_gather(q)
        start_scatter(j + q, q)
      for q in range(NBUF):
        wait_scatter(j + q, q)
        jn = j + NBUF + q

        @pl.when(jn < NCH)
        def _():
          start_gather(jn, q)
      return carry

    lax.fori_loop(0, NCH // NBUF, body, 0)
    for q in range(NCH % NBUF):
      j = (NCH // NBUF) * NBUF + q
      wait_gather(q)
      start_scatter(j, q)
    for q in range(NCH % NBUF):
      j = (NCH // NBUF) * NBUF + q
      wait_scatter(j, q)
    plsc.subcore_barrier()

    @pl.when(s < 15)
    def _():
      pltpu.sync_copy(acc.at[pl.ds(rbase, R_BIG)],
                      out_hbm.at[pl.ds(c * N_NODES + rbase, R_BIG)])

    @pl.when(s == 15)
    def _():
      pltpu.sync_copy(acc.at[pl.ds(rbase, R_LAST)],
                      out_hbm.at[pl.ds(c * N_NODES + rbase, R_LAST)])

  return agg


_agg128 = _make_agg(D_HID)
_agg64 = _make_agg(D_OUT)


def _make_deg():
  mesh = plsc.VectorSubcoreMesh(core_axis_name="c", subcore_axis_name="s")

  @functools.partial(
      pl.kernel, mesh=mesh,
      out_type=jax.ShapeDtypeStruct((2 * N_NODES, DEG_D), jnp.float32),
      compiler_params=pltpu.CompilerParams(use_tc_tiling_on_sc=False),
      scratch_types=[
          pltpu.VMEM((NCH, K), jnp.int32),
          pltpu.VMEM((K, DEG_D), jnp.float32),
          [pltpu.SemaphoreType.DMA for _ in range(NBUF_DEG)],
          pltpu.VMEM_SHARED((N_NODES, DEG_D), jnp.float32),
      ])
  def deg(dst_hbm, ones_hbm, z_hbm, out_hbm, dst_v, ones_v, ssems, acc):
    c = lax.axis_index("c")
    s = lax.axis_index("s")
    wid = s * NC + c
    rbase = s * R_BIG

    @pl.when(s < 15)
    def _():
      pltpu.sync_copy(z_hbm.at[pl.ds(rbase, R_BIG)],
                      acc.at[pl.ds(rbase, R_BIG)])

    @pl.when(s == 15)
    def _():
      pltpu.sync_copy(z_hbm.at[pl.ds(rbase, R_LAST)],
                      acc.at[pl.ds(rbase, R_LAST)])

    pltpu.sync_copy(dst_hbm.at[wid], dst_v)
    pltpu.sync_copy(ones_hbm, ones_v)
    plsc.subcore_barrier()

    def start_scatter(j, b):
      pltpu.async_copy(ones_v, acc.at[dst_v.at[j]], ssems[b], add=True)

    def wait_scatter(j, b):
      pltpu.make_async_copy(ones_v, acc.at[dst_v.at[j]], ssems[b]).wait()

    for q in range(NBUF_DEG):
      start_scatter(q, q)

    def body(i, carry):
      j = NBUF_DEG * i
      for q in range(NBUF_DEG):
        wait_scatter(j + q, q)
        jn = j + NBUF_DEG + q

        @pl.when(jn < NCH)
        def _():
          start_scatter(jn, q)
      return carry

    lax.fori_loop(0, NCH // NBUF_DEG, body, 0)
    for q in range(NCH % NBUF_DEG):
      wait_scatter((NCH // NBUF_DEG) * NBUF_DEG + q, q)
    plsc.subcore_barrier()

    @pl.when(s < 15)
    def _():
      pltpu.sync_copy(acc.at[pl.ds(rbase, R_BIG)],
                      out_hbm.at[pl.ds(c * N_NODES + rbase, R_BIG)])

    @pl.when(s == 15)
    def _():
      pltpu.sync_copy(acc.at[pl.ds(rbase, R_LAST)],
                      out_hbm.at[pl.ds(c * N_NODES + rbase, R_LAST)])

  return deg


_deg = _make_deg()


def _dinv_block(deg_ref):
  d = deg_ref[0, :, 0] + deg_ref[1, :, 0] + 1.0
  return lax.rsqrt(d)[:, None]


def _mm1_body(x_ref, w_ref, deg_ref, o_ref):
  h = jnp.dot(x_ref[...], w_ref[...], preferred_element_type=jnp.float32)
  o_ref[...] = h * _dinv_block(deg_ref)


def _mid_body(p_ref, h_ref, deg_ref, b_ref, w_ref, o_ref):
  dinv = _dinv_block(deg_ref)
  agg = p_ref[0] + p_ref[1] + h_ref[...]
  z = jnp.maximum(agg * dinv + b_ref[...], 0.0)
  o_ref[...] = jnp.dot(z * dinv, w_ref[...], preferred_element_type=jnp.float32)


def _fin_body(q_ref, h_ref, deg_ref, b_ref, o_ref):
  dinv = _dinv_block(deg_ref)
  o_ref[...] = (q_ref[0] + q_ref[1] + h_ref[...]) * dinv + b_ref[...]


def _mm1(x, W1, deg_raw):
  return pl.pallas_call(
      _mm1_body,
      grid=(N_NODES // BM,),
      in_specs=[
          pl.BlockSpec((BM, D_IN), lambda i: (i, 0)),
          pl.BlockSpec((D_IN, D_HID), lambda i: (0, 0)),
          pl.BlockSpec((2, BM, DEG_D), lambda i: (0, i, 0)),
      ],
      out_specs=pl.BlockSpec((BM, D_HID), lambda i: (i, 0)),
      out_shape=jax.ShapeDtypeStruct((N_NODES, D_HID), jnp.float32),
  )(x, W1, deg_raw)


def _mid(p1, h1s, deg_raw, b1, W2):
  return pl.pallas_call(
      _mid_body,
      grid=(N_NODES // BM,),
      in_specs=[
          pl.BlockSpec((2, BM, D_HID), lambda i: (0, i, 0)),
          pl.BlockSpec((BM, D_HID), lambda i: (i, 0)),
          pl.BlockSpec((2, BM, DEG_D), lambda i: (0, i, 0)),
          pl.BlockSpec((1, D_HID), lambda i: (0, 0)),
          pl.BlockSpec((D_HID, D_OUT), lambda i: (0, 0)),
      ],
      out_specs=pl.BlockSpec((BM, D_OUT), lambda i: (i, 0)),
      out_shape=jax.ShapeDtypeStruct((N_NODES, D_OUT), jnp.float32),
  )(p1, h1s, deg_raw, b1, W2)


def _fin(q, h2s, deg_raw, b2):
  return pl.pallas_call(
      _fin_body,
      grid=(N_NODES // BM,),
      in_specs=[
          pl.BlockSpec((2, BM, D_OUT), lambda i: (0, i, 0)),
          pl.BlockSpec((BM, D_OUT), lambda i: (i, 0)),
          pl.BlockSpec((2, BM, DEG_D), lambda i: (0, i, 0)),
          pl.BlockSpec((1, D_OUT), lambda i: (0, 0)),
      ],
      out_specs=pl.BlockSpec((BM, D_OUT), lambda i: (i, 0)),
      out_shape=jax.ShapeDtypeStruct((N_NODES, D_OUT), jnp.float32),
  )(q, h2s, deg_raw, b2)


def kernel(x, edge_index, W1, b1, W2, b2):
  src = edge_index[0].reshape(NW, NCH, K)
  dst = edge_index[1].reshape(NW, NCH, K)
  ones16 = jnp.ones((K, DEG_D), jnp.float32)
  z16 = jnp.zeros((N_NODES, DEG_D), jnp.float32)
  z128 = jnp.zeros((N_NODES, D_HID), jnp.float32)
  z64 = jnp.zeros((N_NODES, D_OUT), jnp.float32)

  deg_raw = _deg(dst, ones16, z16).reshape(2, N_NODES, DEG_D)
  h1s = _mm1(x, W1, deg_raw)
  p1 = _agg128(h1s, src, dst, z128).reshape(2, N_NODES, D_HID)
  h2s = _mid(p1, h1s, deg_raw, b1.reshape(1, D_HID), W2)
  p2 = _agg64(h2s, src, dst, z64).reshape(2, N_NODES, D_OUT)
  return _fin(p2, h2s, deg_raw, b2.reshape(1, D_OUT))


# speedup vs baseline: 31.3778x; 1.1103x over previous
import functools

import jax
import jax.numpy as jnp
from jax import lax
from jax.experimental import pallas as pl
from jax.experimental.pallas import tpu as pltpu
from jax.experimental.pallas import tpu_sc as plsc

N_NODES = 10000
N_EDGES = 320000
D_IN = 128
D_HID = 128
D_OUT = 64

NC = 2
NS = 16
NW = NC * NS
E_PER_W = N_EDGES // NW
K = 80
NCH = E_PER_W // K
R_BIG = 640
R_LAST = N_NODES - 15 * R_BIG
DEG_D = 16
NBUF_DEG = 4
BM = 1000


def _make_agg(D, NBUF):
  mesh = plsc.VectorSubcoreMesh(core_axis_name="c", subcore_axis_name="s")

  @functools.partial(
      pl.kernel, mesh=mesh,
      out_type=jax.ShapeDtypeStruct((2 * N_NODES, D), jnp.float32),
      compiler_params=pltpu.CompilerParams(use_tc_tiling_on_sc=False),
      scratch_types=[
          pltpu.VMEM((NCH, K), jnp.int32),
          pltpu.VMEM((NCH, K), jnp.int32),
          [pltpu.VMEM((K, D), jnp.float32) for _ in range(NBUF)],
          [pltpu.SemaphoreType.DMA for _ in range(NBUF)],
          [pltpu.SemaphoreType.DMA for _ in range(NBUF)],
          pltpu.VMEM_SHARED((N_NODES, D), jnp.float32),
      ])
  def agg(h_hbm, src_hbm, dst_hbm, z_hbm, out_hbm, src_v, dst_v, rows, gsems,
          ssems, acc):
    c = lax.axis_index("c")
    s = lax.axis_index("s")
    wid = s * NC + c
    rbase = s * R_BIG
    @pl.when(s < 15)
    def _():
      pltpu.sync_copy(z_hbm.at[pl.ds(rbase, R_BIG)],
                      acc.at[pl.ds(rbase, R_BIG)])

    @pl.when(s == 15)
    def _():
      pltpu.sync_copy(z_hbm.at[pl.ds(rbase, R_LAST)],
                      acc.at[pl.ds(rbase, R_LAST)])

    pltpu.sync_copy(src_hbm.at[wid], src_v)
    pltpu.sync_copy(dst_hbm.at[wid], dst_v)
    plsc.subcore_barrier()

    def start_gather(j, b):
      pltpu.async_copy(h_hbm.at[src_v.at[j]], rows[b], gsems[b])

    def wait_gather(b):
      pltpu.make_async_copy(h_hbm.at[src_v.at[0]], rows[b], gsems[b]).wait()

    def start_scatter(j, b):
      pltpu.async_copy(rows[b], acc.at[dst_v.at[j]], ssems[b], add=True)

    def wait_scatter(j, b):
      pltpu.make_async_copy(rows[b], acc.at[dst_v.at[j]], ssems[b]).wait()

    for q in range(NBUF):
      start_gather(q, q)

    def body(i, carry):
      j = NBUF * i
      for q in range(NBUF):
        wait_gather(q)
        start_scatter(j + q, q)
      for q in range(NBUF):
        wait_scatter(j + q, q)
        jn = j + NBUF + q

        @pl.when(jn < NCH)
        def _():
          start_gather(jn, q)
      return carry

    lax.fori_loop(0, NCH // NBUF, body, 0)
    for q in range(NCH % NBUF):
      j = (NCH // NBUF) * NBUF + q
      wait_gather(q)
      start_scatter(j, q)
    for q in range(NCH % NBUF):
      j = (NCH // NBUF) * NBUF + q
      wait_scatter(j, q)
    plsc.subcore_barrier()

    @pl.when(s < 15)
    def _():
      pltpu.sync_copy(acc.at[pl.ds(rbase, R_BIG)],
                      out_hbm.at[pl.ds(c * N_NODES + rbase, R_BIG)])

    @pl.when(s == 15)
    def _():
      pltpu.sync_copy(acc.at[pl.ds(rbase, R_LAST)],
                      out_hbm.at[pl.ds(c * N_NODES + rbase, R_LAST)])

  return agg


_agg128 = _make_agg(D_HID, 2)
_agg64 = _make_agg(D_OUT, 6)


def _make_deg():
  mesh = plsc.VectorSubcoreMesh(core_axis_name="c", subcore_axis_name="s")

  @functools.partial(
      pl.kernel, mesh=mesh,
      out_type=jax.ShapeDtypeStruct((2 * N_NODES, DEG_D), jnp.float32),
      compiler_params=pltpu.CompilerParams(use_tc_tiling_on_sc=False),
      scratch_types=[
          pltpu.VMEM((NCH, K), jnp.int32),
          pltpu.VMEM((K, DEG_D), jnp.float32),
          [pltpu.SemaphoreType.DMA for _ in range(NBUF_DEG)],
          pltpu.VMEM_SHARED((N_NODES, DEG_D), jnp.float32),
      ])
  def deg(dst_hbm, ones_hbm, z_hbm, out_hbm, dst_v, ones_v, ssems, acc):
    c = lax.axis_index("c")
    s = lax.axis_index("s")
    wid = s * NC + c
    rbase = s * R_BIG

    @pl.when(s < 15)
    def _():
      pltpu.sync_copy(z_hbm.at[pl.ds(rbase, R_BIG)],
                      acc.at[pl.ds(rbase, R_BIG)])

    @pl.when(s == 15)
    def _():
      pltpu.sync_copy(z_hbm.at[pl.ds(rbase, R_LAST)],
                      acc.at[pl.ds(rbase, R_LAST)])

    pltpu.sync_copy(dst_hbm.at[wid], dst_v)
    pltpu.sync_copy(ones_hbm, ones_v)
    plsc.subcore_barrier()

    def start_scatter(j, b):
      pltpu.async_copy(ones_v, acc.at[dst_v.at[j]], ssems[b], add=True)

    def wait_scatter(j, b):
      pltpu.make_async_copy(ones_v, acc.at[dst_v.at[j]], ssems[b]).wait()

    for q in range(NBUF_DEG):
      start_scatter(q, q)

    def body(i, carry):
      j = NBUF_DEG * i
      for q in range(NBUF_DEG):
        wait_scatter(j + q, q)
        jn = j + NBUF_DEG + q

        @pl.when(jn < NCH)
        def _():
          start_scatter(jn, q)
      return carry

    lax.fori_loop(0, NCH // NBUF_DEG, body, 0)
    for q in range(NCH % NBUF_DEG):
      wait_scatter((NCH // NBUF_DEG) * NBUF_DEG + q, q)
    plsc.subcore_barrier()

    @pl.when(s < 15)
    def _():
      pltpu.sync_copy(acc.at[pl.ds(rbase, R_BIG)],
                      out_hbm.at[pl.ds(c * N_NODES + rbase, R_BIG)])

    @pl.when(s == 15)
    def _():
      pltpu.sync_copy(acc.at[pl.ds(rbase, R_LAST)],
                      out_hbm.at[pl.ds(c * N_NODES + rbase, R_LAST)])

  return deg


_deg = _make_deg()


def _dinv_block(deg_ref):
  d = deg_ref[0, :, 0] + deg_ref[1, :, 0] + 1.0
  return lax.rsqrt(d)[:, None]


def _mm1_body(x_ref, w_ref, deg_ref, o_ref):
  h = jnp.dot(x_ref[...], w_ref[...], preferred_element_type=jnp.float32)
  o_ref[...] = h * _dinv_block(deg_ref)


def _mid_body(p_ref, h_ref, deg_ref, b_ref, w_ref, o_ref):
  dinv = _dinv_block(deg_ref)
  agg = p_ref[0] + p_ref[1] + h_ref[...]
  z = jnp.maximum(agg * dinv + b_ref[...], 0.0)
  o_ref[...] = jnp.dot(z * dinv, w_ref[...], preferred_element_type=jnp.float32)


def _fin_body(q_ref, h_ref, deg_ref, b_ref, o_ref):
  dinv = _dinv_block(deg_ref)
  o_ref[...] = (q_ref[0] + q_ref[1] + h_ref[...]) * dinv + b_ref[...]


def _mm1(x, W1, deg_raw):
  return pl.pallas_call(
      _mm1_body,
      grid=(N_NODES // BM,),
      in_specs=[
          pl.BlockSpec((BM, D_IN), lambda i: (i, 0)),
          pl.BlockSpec((D_IN, D_HID), lambda i: (0, 0)),
          pl.BlockSpec((2, BM, DEG_D), lambda i: (0, i, 0)),
      ],
      out_specs=pl.BlockSpec((BM, D_HID), lambda i: (i, 0)),
      out_shape=jax.ShapeDtypeStruct((N_NODES, D_HID), jnp.float32),
  )(x, W1, deg_raw)


def _mid(p1, h1s, deg_raw, b1, W2):
  return pl.pallas_call(
      _mid_body,
      grid=(N_NODES // BM,),
      in_specs=[
          pl.BlockSpec((2, BM, D_HID), lambda i: (0, i, 0)),
          pl.BlockSpec((BM, D_HID), lambda i: (i, 0)),
          pl.BlockSpec((2, BM, DEG_D), lambda i: (0, i, 0)),
          pl.BlockSpec((1, D_HID), lambda i: (0, 0)),
          pl.BlockSpec((D_HID, D_OUT), lambda i: (0, 0)),
      ],
      out_specs=pl.BlockSpec((BM, D_OUT), lambda i: (i, 0)),
      out_shape=jax.ShapeDtypeStruct((N_NODES, D_OUT), jnp.float32),
  )(p1, h1s, deg_raw, b1, W2)


def _fin(q, h2s, deg_raw, b2):
  return pl.pallas_call(
      _fin_body,
      grid=(N_NODES // BM,),
      in_specs=[
          pl.BlockSpec((2, BM, D_OUT), lambda i: (0, i, 0)),
          pl.BlockSpec((BM, D_OUT), lambda i: (i, 0)),
          pl.BlockSpec((2, BM, DEG_D), lambda i: (0, i, 0)),
          pl.BlockSpec((1, D_OUT), lambda i: (0, 0)),
      ],
      out_specs=pl.BlockSpec((BM, D_OUT), lambda i: (i, 0)),
      out_shape=jax.ShapeDtypeStruct((N_NODES, D_OUT), jnp.float32),
  )(q, h2s, deg_raw, b2)


def kernel(x, edge_index, W1, b1, W2, b2):
  src = edge_index[0].reshape(NW, NCH, K)
  dst = edge_index[1].reshape(NW, NCH, K)
  ones16 = jnp.ones((K, DEG_D), jnp.float32)
  z16 = jnp.zeros((N_NODES, DEG_D), jnp.float32)
  z128 = jnp.zeros((N_NODES, D_HID), jnp.float32)
  z64 = jnp.zeros((N_NODES, D_OUT), jnp.float32)

  deg_raw = _deg(dst, ones16, z16).reshape(2, N_NODES, DEG_D)
  h1s = _mm1(x, W1, deg_raw)
  p1 = _agg128(h1s, src, dst, z128).reshape(2, N_NODES, D_HID)
  h2s = _mid(p1, h1s, deg_raw, b1.reshape(1, D_HID), W2)
  p2 = _agg64(h2s, src, dst, z64).reshape(2, N_NODES, D_OUT)
  return _fin(p2, h2s, deg_raw, b2.reshape(1, D_OUT))


# speedup vs baseline: 32.5762x; 1.0382x over previous
import functools

import jax
import jax.numpy as jnp
from jax import lax
from jax.experimental import pallas as pl
from jax.experimental.pallas import tpu as pltpu
from jax.experimental.pallas import tpu_sc as plsc

N_NODES = 10000
N_EDGES = 320000
D_IN = 128
D_HID = 128
D_OUT = 64

NC = 2
NS = 16
NW = NC * NS
E_PER_W = N_EDGES // NW
K = 80
NCH = E_PER_W // K
R_BIG = 640
R_LAST = N_NODES - 15 * R_BIG
DEG_D = 16
NBUF_DEG = 4
BM = 1000


def _make_agg(D, NBUF):
  mesh = plsc.VectorSubcoreMesh(core_axis_name="c", subcore_axis_name="s")

  @functools.partial(
      pl.kernel, mesh=mesh,
      out_type=jax.ShapeDtypeStruct((2, N_NODES, D), jnp.float32),
      compiler_params=pltpu.CompilerParams(use_tc_tiling_on_sc=False),
      scratch_types=[
          pltpu.VMEM((NCH, K), jnp.int32),
          pltpu.VMEM((NCH, K), jnp.int32),
          [pltpu.VMEM((K, D), jnp.float32) for _ in range(NBUF)],
          [pltpu.SemaphoreType.DMA for _ in range(NBUF)],
          [pltpu.SemaphoreType.DMA for _ in range(NBUF)],
          pltpu.VMEM_SHARED((N_NODES, D), jnp.float32),
      ])
  def agg(h_hbm, eidx_hbm, z_hbm, out_hbm, src_v, dst_v, rows, gsems,
          ssems, acc):
    c = lax.axis_index("c")
    s = lax.axis_index("s")
    wid = s * NC + c
    rbase = s * R_BIG
    @pl.when(s < 15)
    def _():
      pltpu.sync_copy(z_hbm.at[pl.ds(rbase, R_BIG)],
                      acc.at[pl.ds(rbase, R_BIG)])

    @pl.when(s == 15)
    def _():
      pltpu.sync_copy(z_hbm.at[pl.ds(rbase, R_LAST)],
                      acc.at[pl.ds(rbase, R_LAST)])

    pltpu.sync_copy(eidx_hbm.at[0, wid], src_v)
    pltpu.sync_copy(eidx_hbm.at[1, wid], dst_v)
    plsc.subcore_barrier()

    def start_gather(j, b):
      pltpu.async_copy(h_hbm.at[src_v.at[j]], rows[b], gsems[b])

    def wait_gather(b):
      pltpu.make_async_copy(h_hbm.at[src_v.at[0]], rows[b], gsems[b]).wait()

    def start_scatter(j, b):
      pltpu.async_copy(rows[b], acc.at[dst_v.at[j]], ssems[b], add=True)

    def wait_scatter(j, b):
      pltpu.make_async_copy(rows[b], acc.at[dst_v.at[j]], ssems[b]).wait()

    for q in range(NBUF):
      start_gather(q, q)

    def body(i, carry):
      j = NBUF * i
      for q in range(NBUF):
        wait_gather(q)
        start_scatter(j + q, q)
      for q in range(NBUF):
        wait_scatter(j + q, q)
        jn = j + NBUF + q

        @pl.when(jn < NCH)
        def _():
          start_gather(jn, q)
      return carry

    lax.fori_loop(0, NCH // NBUF, body, 0)
    for q in range(NCH % NBUF):
      j = (NCH // NBUF) * NBUF + q
      wait_gather(q)
      start_scatter(j, q)
    for q in range(NCH % NBUF):
      j = (NCH // NBUF) * NBUF + q
      wait_scatter(j, q)
    plsc.subcore_barrier()

    @pl.when(s < 15)
    def _():
      pltpu.sync_copy(acc.at[pl.ds(rbase, R_BIG)],
                      out_hbm.at[c, pl.ds(rbase, R_BIG)])

    @pl.when(s == 15)
    def _():
      pltpu.sync_copy(acc.at[pl.ds(rbase, R_LAST)],
                      out_hbm.at[c, pl.ds(rbase, R_LAST)])

  return agg


_agg128 = _make_agg(D_HID, 2)
_agg64 = _make_agg(D_OUT, 6)


def _make_deg():
  mesh = plsc.VectorSubcoreMesh(core_axis_name="c", subcore_axis_name="s")

  @functools.partial(
      pl.kernel, mesh=mesh,
      out_type=jax.ShapeDtypeStruct((2, N_NODES, DEG_D), jnp.float32),
      compiler_params=pltpu.CompilerParams(use_tc_tiling_on_sc=False),
      scratch_types=[
          pltpu.VMEM((NCH, K), jnp.int32),
          pltpu.VMEM((K, DEG_D), jnp.float32),
          [pltpu.SemaphoreType.DMA for _ in range(NBUF_DEG)],
          pltpu.VMEM_SHARED((N_NODES, DEG_D), jnp.float32),
      ])
  def deg(eidx_hbm, ones_hbm, z_hbm, out_hbm, dst_v, ones_v, ssems, acc):
    c = lax.axis_index("c")
    s = lax.axis_index("s")
    wid = s * NC + c
    rbase = s * R_BIG

    @pl.when(s < 15)
    def _():
      pltpu.sync_copy(z_hbm.at[pl.ds(rbase, R_BIG)],
                      acc.at[pl.ds(rbase, R_BIG)])

    @pl.when(s == 15)
    def _():
      pltpu.sync_copy(z_hbm.at[pl.ds(rbase, R_LAST)],
                      acc.at[pl.ds(rbase, R_LAST)])

    pltpu.sync_copy(eidx_hbm.at[1, wid], dst_v)
    pltpu.sync_copy(ones_hbm, ones_v)
    plsc.subcore_barrier()

    def start_scatter(j, b):
      pltpu.async_copy(ones_v, acc.at[dst_v.at[j]], ssems[b], add=True)

    def wait_scatter(j, b):
      pltpu.make_async_copy(ones_v, acc.at[dst_v.at[j]], ssems[b]).wait()

    for q in range(NBUF_DEG):
      start_scatter(q, q)

    def body(i, carry):
      j = NBUF_DEG * i
      for q in range(NBUF_DEG):
        wait_scatter(j + q, q)
        jn = j + NBUF_DEG + q

        @pl.when(jn < NCH)
        def _():
          start_scatter(jn, q)
      return carry

    lax.fori_loop(0, NCH // NBUF_DEG, body, 0)
    for q in range(NCH % NBUF_DEG):
      wait_scatter((NCH // NBUF_DEG) * NBUF_DEG + q, q)
    plsc.subcore_barrier()

    @pl.when(s < 15)
    def _():
      pltpu.sync_copy(acc.at[pl.ds(rbase, R_BIG)],
                      out_hbm.at[c, pl.ds(rbase, R_BIG)])

    @pl.when(s == 15)
    def _():
      pltpu.sync_copy(acc.at[pl.ds(rbase, R_LAST)],
                      out_hbm.at[c, pl.ds(rbase, R_LAST)])

  return deg


_deg = _make_deg()


def _mm1_body(x_ref, w_ref, deg_ref, o_ref, dinv_ref):
  d = deg_ref[0, :, 0] + deg_ref[1, :, 0] + 1.0
  dinv = lax.rsqrt(d)[:, None]
  dinv_ref[...] = jnp.broadcast_to(dinv, dinv_ref.shape)
  h = jnp.dot(x_ref[...], w_ref[...], preferred_element_type=jnp.float32)
  o_ref[...] = h * dinv


def _mid_body(p_ref, h_ref, dinv_ref, b_ref, w_ref, o_ref):
  dinv = dinv_ref[:, 0][:, None]
  agg = p_ref[0] + p_ref[1] + h_ref[...]
  z = jnp.maximum(agg * dinv + b_ref[...], 0.0)
  o_ref[...] = jnp.dot(z * dinv, w_ref[...], preferred_element_type=jnp.float32)


def _fin_body(q_ref, h_ref, dinv_ref, b_ref, o_ref):
  dinv = dinv_ref[:, 0][:, None]
  o_ref[...] = (q_ref[0] + q_ref[1] + h_ref[...]) * dinv + b_ref[...]


def _mm1(x, W1, deg_raw):
  return pl.pallas_call(
      _mm1_body,
      grid=(N_NODES // BM,),
      in_specs=[
          pl.BlockSpec((BM, D_IN), lambda i: (i, 0)),
          pl.BlockSpec((D_IN, D_HID), lambda i: (0, 0)),
          pl.BlockSpec((2, BM, DEG_D), lambda i: (0, i, 0)),
      ],
      out_specs=[
          pl.BlockSpec((BM, D_HID), lambda i: (i, 0)),
          pl.BlockSpec((BM, 8), lambda i: (i, 0)),
      ],
      out_shape=[
          jax.ShapeDtypeStruct((N_NODES, D_HID), jnp.float32),
          jax.ShapeDtypeStruct((N_NODES, 8), jnp.float32),
      ],
  )(x, W1, deg_raw)


def _mid(p1, h1s, dinv, b1, W2):
  return pl.pallas_call(
      _mid_body,
      grid=(N_NODES // BM,),
      in_specs=[
          pl.BlockSpec((2, BM, D_HID), lambda i: (0, i, 0)),
          pl.BlockSpec((BM, D_HID), lambda i: (i, 0)),
          pl.BlockSpec((BM, 8), lambda i: (i, 0)),
          pl.BlockSpec((1, D_HID), lambda i: (0, 0)),
          pl.BlockSpec((D_HID, D_OUT), lambda i: (0, 0)),
      ],
      out_specs=pl.BlockSpec((BM, D_OUT), lambda i: (i, 0)),
      out_shape=jax.ShapeDtypeStruct((N_NODES, D_OUT), jnp.float32),
  )(p1, h1s, dinv, b1, W2)


def _fin(q, h2s, dinv, b2):
  return pl.pallas_call(
      _fin_body,
      grid=(N_NODES // BM,),
      in_specs=[
          pl.BlockSpec((2, BM, D_OUT), lambda i: (0, i, 0)),
          pl.BlockSpec((BM, D_OUT), lambda i: (i, 0)),
          pl.BlockSpec((BM, 8), lambda i: (i, 0)),
          pl.BlockSpec((1, D_OUT), lambda i: (0, 0)),
      ],
      out_specs=pl.BlockSpec((BM, D_OUT), lambda i: (i, 0)),
      out_shape=jax.ShapeDtypeStruct((N_NODES, D_OUT), jnp.float32),
  )(q, h2s, dinv, b2)


def kernel(x, edge_index, W1, b1, W2, b2):
  eidx = edge_index.reshape(2, NW, NCH, K)
  ones16 = jnp.ones((K, DEG_D), jnp.float32)
  z16 = jnp.zeros((N_NODES, DEG_D), jnp.float32)
  z128 = jnp.zeros((N_NODES, D_HID), jnp.float32)
  z64 = jnp.zeros((N_NODES, D_OUT), jnp.float32)

  deg_raw = _deg(eidx, ones16, z16)
  h1s, dinv = _mm1(x, W1, deg_raw)
  p1 = _agg128(h1s, eidx, z128)
  h2s = _mid(p1, h1s, dinv, b1.reshape(1, D_HID), W2)
  p2 = _agg64(h2s, eidx, z64)
  return _fin(p2, h2s, dinv, b2.reshape(1, D_OUT))


# speedup vs baseline: 36.0450x; 1.1065x over previous
import functools

import jax
import jax.numpy as jnp
from jax import lax
from jax.experimental import pallas as pl
from jax.experimental.pallas import tpu as pltpu
from jax.experimental.pallas import tpu_sc as plsc

N_NODES = 10000
N_EDGES = 320000
D_IN = 128
D_HID = 128
D_OUT = 64

NC = 2
NS = 16
NW = NC * NS
E_PER_W = N_EDGES // NW
K = 80
NCH = E_PER_W // K
R_BIG = 640
R_LAST = N_NODES - 15 * R_BIG
DEG_D = 16
NBUF_DEG = 4
BM = 1000


def _make_agg(D, NBUF, SEG):
  assert NCH % SEG == 0 and SEG >= NBUF
  mesh = plsc.VectorSubcoreMesh(core_axis_name="c", subcore_axis_name="s")

  @functools.partial(
      pl.kernel, mesh=mesh,
      out_type=jax.ShapeDtypeStruct((2, N_NODES, D), jnp.float32),
      compiler_params=pltpu.CompilerParams(use_tc_tiling_on_sc=False),
      scratch_types=[
          pltpu.VMEM((SEG, K), jnp.int32),
          pltpu.VMEM((SEG, K), jnp.int32),
          [pltpu.VMEM((K, D), jnp.float32) for _ in range(NBUF)],
          [pltpu.SemaphoreType.DMA for _ in range(NBUF)],
          [pltpu.SemaphoreType.DMA for _ in range(NBUF)],
          pltpu.VMEM_SHARED((N_NODES, D), jnp.float32),
      ])
  def agg(h_hbm, eidx_hbm, z_hbm, out_hbm, src_v, dst_v, rows, gsems,
          ssems, acc):
    c = lax.axis_index("c")
    s = lax.axis_index("s")
    wid = s * NC + c
    rbase = s * R_BIG
    @pl.when(s < 15)
    def _():
      pltpu.sync_copy(z_hbm.at[pl.ds(rbase, R_BIG)],
                      acc.at[pl.ds(rbase, R_BIG)])

    @pl.when(s == 15)
    def _():
      pltpu.sync_copy(z_hbm.at[pl.ds(rbase, R_LAST)],
                      acc.at[pl.ds(rbase, R_LAST)])

    plsc.subcore_barrier()

    def start_gather(j, b):
      pltpu.async_copy(h_hbm.at[src_v.at[j]], rows[b], gsems[b])

    def wait_gather(b):
      pltpu.make_async_copy(h_hbm.at[src_v.at[0]], rows[b], gsems[b]).wait()

    def start_scatter(j, b):
      pltpu.async_copy(rows[b], acc.at[dst_v.at[j]], ssems[b], add=True)

    def wait_scatter(j, b):
      pltpu.make_async_copy(rows[b], acc.at[dst_v.at[j]], ssems[b]).wait()

    def body(i, carry):
      j = NBUF * i
      for q in range(NBUF):
        wait_gather(q)
        start_scatter(j + q, q)
      for q in range(NBUF):
        wait_scatter(j + q, q)
        jn = j + NBUF + q

        @pl.when(jn < SEG)
        def _():
          start_gather(jn, q)
      return carry

    for seg in range(NCH // SEG):
      pltpu.sync_copy(eidx_hbm.at[0, wid, pl.ds(seg * SEG, SEG)], src_v)
      pltpu.sync_copy(eidx_hbm.at[1, wid, pl.ds(seg * SEG, SEG)], dst_v)
      for q in range(NBUF):
        start_gather(q, q)
      lax.fori_loop(0, SEG // NBUF, body, 0)
      for q in range(SEG % NBUF):
        j = (SEG // NBUF) * NBUF + q
        wait_gather(q)
        start_scatter(j, q)
      for q in range(SEG % NBUF):
        j = (SEG // NBUF) * NBUF + q
        wait_scatter(j, q)
    plsc.subcore_barrier()

    @pl.when(s < 15)
    def _():
      pltpu.sync_copy(acc.at[pl.ds(rbase, R_BIG)],
                      out_hbm.at[c, pl.ds(rbase, R_BIG)])

    @pl.when(s == 15)
    def _():
      pltpu.sync_copy(acc.at[pl.ds(rbase, R_LAST)],
                      out_hbm.at[c, pl.ds(rbase, R_LAST)])

  return agg


_agg128 = _make_agg(D_HID, 4, 25)
_agg64 = _make_agg(D_OUT, 6, NCH)


def _make_deg():
  mesh = plsc.VectorSubcoreMesh(core_axis_name="c", subcore_axis_name="s")

  @functools.partial(
      pl.kernel, mesh=mesh,
      out_type=jax.ShapeDtypeStruct((2, N_NODES, DEG_D), jnp.float32),
      compiler_params=pltpu.CompilerParams(use_tc_tiling_on_sc=False),
      scratch_types=[
          pltpu.VMEM((NCH, K), jnp.int32),
          pltpu.VMEM((K, DEG_D), jnp.float32),
          [pltpu.SemaphoreType.DMA for _ in range(NBUF_DEG)],
          pltpu.VMEM_SHARED((N_NODES, DEG_D), jnp.float32),
      ])
  def deg(eidx_hbm, ones_hbm, z_hbm, out_hbm, dst_v, ones_v, ssems, acc):
    c = lax.axis_index("c")
    s = lax.axis_index("s")
    wid = s * NC + c
    rbase = s * R_BIG

    @pl.when(s < 15)
    def _():
      pltpu.sync_copy(z_hbm.at[pl.ds(rbase, R_BIG)],
                      acc.at[pl.ds(rbase, R_BIG)])

    @pl.when(s == 15)
    def _():
      pltpu.sync_copy(z_hbm.at[pl.ds(rbase, R_LAST)],
                      acc.at[pl.ds(rbase, R_LAST)])

    pltpu.sync_copy(eidx_hbm.at[1, wid], dst_v)
    pltpu.sync_copy(ones_hbm, ones_v)
    plsc.subcore_barrier()

    def start_scatter(j, b):
      pltpu.async_copy(ones_v, acc.at[dst_v.at[j]], ssems[b], add=True)

    def wait_scatter(j, b):
      pltpu.make_async_copy(ones_v, acc.at[dst_v.at[j]], ssems[b]).wait()

    for q in range(NBUF_DEG):
      start_scatter(q, q)

    def body(i, carry):
      j = NBUF_DEG * i
      for q in range(NBUF_DEG):
        wait_scatter(j + q, q)
        jn = j + NBUF_DEG + q

        @pl.when(jn < NCH)
        def _():
          start_scatter(jn, q)
      return carry

    lax.fori_loop(0, NCH // NBUF_DEG, body, 0)
    for q in range(NCH % NBUF_DEG):
      wait_scatter((NCH // NBUF_DEG) * NBUF_DEG + q, q)
    plsc.subcore_barrier()

    @pl.when(s < 15)
    def _():
      pltpu.sync_copy(acc.at[pl.ds(rbase, R_BIG)],
                      out_hbm.at[c, pl.ds(rbase, R_BIG)])

    @pl.when(s == 15)
    def _():
      pltpu.sync_copy(acc.at[pl.ds(rbase, R_LAST)],
                      out_hbm.at[c, pl.ds(rbase, R_LAST)])

  return deg


_deg = _make_deg()


def _mm1_body(x_ref, w_ref, deg_ref, o_ref, dinv_ref):
  d = deg_ref[0, :, 0] + deg_ref[1, :, 0] + 1.0
  dinv = lax.rsqrt(d)[:, None]
  dinv_ref[...] = jnp.broadcast_to(dinv, dinv_ref.shape)
  h = jnp.dot(x_ref[...], w_ref[...], preferred_element_type=jnp.float32)
  o_ref[...] = h * dinv


def _mid_body(p_ref, h_ref, dinv_ref, b_ref, w_ref, o_ref):
  dinv = dinv_ref[:, 0][:, None]
  agg = p_ref[0] + p_ref[1] + h_ref[...]
  z = jnp.maximum(agg * dinv + b_ref[...], 0.0)
  o_ref[...] = jnp.dot(z * dinv, w_ref[...], preferred_element_type=jnp.float32)


def _fin_body(q_ref, h_ref, dinv_ref, b_ref, o_ref):
  dinv = dinv_ref[:, 0][:, None]
  o_ref[...] = (q_ref[0] + q_ref[1] + h_ref[...]) * dinv + b_ref[...]


def _mm1(x, W1, deg_raw):
  return pl.pallas_call(
      _mm1_body,
      grid=(N_NODES // BM,),
      in_specs=[
          pl.BlockSpec((BM, D_IN), lambda i: (i, 0)),
          pl.BlockSpec((D_IN, D_HID), lambda i: (0, 0)),
          pl.BlockSpec((2, BM, DEG_D), lambda i: (0, i, 0)),
      ],
      out_specs=[
          pl.BlockSpec((BM, D_HID), lambda i: (i, 0)),
          pl.BlockSpec((BM, 8), lambda i: (i, 0)),
      ],
      out_shape=[
          jax.ShapeDtypeStruct((N_NODES, D_HID), jnp.float32),
          jax.ShapeDtypeStruct((N_NODES, 8), jnp.float32),
      ],
  )(x, W1, deg_raw)


def _mid(p1, h1s, dinv, b1, W2):
  return pl.pallas_call(
      _mid_body,
      grid=(N_NODES // BM,),
      in_specs=[
          pl.BlockSpec((2, BM, D_HID), lambda i: (0, i, 0)),
          pl.BlockSpec((BM, D_HID), lambda i: (i, 0)),
          pl.BlockSpec((BM, 8), lambda i: (i, 0)),
          pl.BlockSpec((1, D_HID), lambda i: (0, 0)),
          pl.BlockSpec((D_HID, D_OUT), lambda i: (0, 0)),
      ],
      out_specs=pl.BlockSpec((BM, D_OUT), lambda i: (i, 0)),
      out_shape=jax.ShapeDtypeStruct((N_NODES, D_OUT), jnp.float32),
  )(p1, h1s, dinv, b1, W2)


def _fin(q, h2s, dinv, b2):
  return pl.pallas_call(
      _fin_body,
      grid=(N_NODES // BM,),
      in_specs=[
          pl.BlockSpec((2, BM, D_OUT), lambda i: (0, i, 0)),
          pl.BlockSpec((BM, D_OUT), lambda i: (i, 0)),
          pl.BlockSpec((BM, 8), lambda i: (i, 0)),
          pl.BlockSpec((1, D_OUT), lambda i: (0, 0)),
      ],
      out_specs=pl.BlockSpec((BM, D_OUT), lambda i: (i, 0)),
      out_shape=jax.ShapeDtypeStruct((N_NODES, D_OUT), jnp.float32),
  )(q, h2s, dinv, b2)


def kernel(x, edge_index, W1, b1, W2, b2):
  eidx = edge_index.reshape(2, NW, NCH, K)
  ones16 = jnp.ones((K, DEG_D), jnp.float32)
  z16 = jnp.zeros((N_NODES, DEG_D), jnp.float32)
  z128 = jnp.zeros((N_NODES, D_HID), jnp.float32)
  z64 = jnp.zeros((N_NODES, D_OUT), jnp.float32)

  deg_raw = _deg(eidx, ones16, z16)
  h1s, dinv = _mm1(x, W1, deg_raw)
  p1 = _agg128(h1s, eidx, z128)
  h2s = _mid(p1, h1s, dinv, b1.reshape(1, D_HID), W2)
  p2 = _agg64(h2s, eidx, z64)
  return _fin(p2, h2s, dinv, b2.reshape(1, D_OUT))


# speedup vs baseline: 36.2604x; 1.0060x over previous
import functools

import jax
import jax.numpy as jnp
from jax import lax
from jax.experimental import pallas as pl
from jax.experimental.pallas import tpu as pltpu
from jax.experimental.pallas import tpu_sc as plsc

N_NODES = 10000
N_EDGES = 320000
D_IN = 128
D_HID = 128
D_OUT = 64

NC = 2
NS = 16
NW = NC * NS
E_PER_W = N_EDGES // NW
K = 80
NCH = E_PER_W // K
R_BIG = 640
R_LAST = N_NODES - 15 * R_BIG
DEG_D = 16
NBUF_DEG = 8
BM = 1000


def _make_agg(D, NBUF, SEG):
  assert NCH % SEG == 0 and SEG >= NBUF
  mesh = plsc.VectorSubcoreMesh(core_axis_name="c", subcore_axis_name="s")

  @functools.partial(
      pl.kernel, mesh=mesh,
      out_type=jax.ShapeDtypeStruct((2, N_NODES, D), jnp.float32),
      compiler_params=pltpu.CompilerParams(use_tc_tiling_on_sc=False),
      scratch_types=[
          pltpu.VMEM((SEG, K), jnp.int32),
          pltpu.VMEM((SEG, K), jnp.int32),
          [pltpu.VMEM((K, D), jnp.float32) for _ in range(NBUF)],
          [pltpu.SemaphoreType.DMA for _ in range(NBUF)],
          [pltpu.SemaphoreType.DMA for _ in range(NBUF)],
          pltpu.VMEM_SHARED((N_NODES, D), jnp.float32),
      ])
  def agg(h_hbm, eidx_hbm, z_hbm, out_hbm, src_v, dst_v, rows, gsems,
          ssems, acc):
    c = lax.axis_index("c")
    s = lax.axis_index("s")
    wid = s * NC + c
    rbase = s * R_BIG
    @pl.when(s < 15)
    def _():
      pltpu.sync_copy(z_hbm.at[pl.ds(rbase, R_BIG)],
                      acc.at[pl.ds(rbase, R_BIG)])

    @pl.when(s == 15)
    def _():
      pltpu.sync_copy(z_hbm.at[pl.ds(rbase, R_LAST)],
                      acc.at[pl.ds(rbase, R_LAST)])

    plsc.subcore_barrier()

    def start_gather(j, b):
      pltpu.async_copy(h_hbm.at[src_v.at[j]], rows[b], gsems[b])

    def wait_gather(b):
      pltpu.make_async_copy(h_hbm.at[src_v.at[0]], rows[b], gsems[b]).wait()

    def start_scatter(j, b):
      pltpu.async_copy(rows[b], acc.at[dst_v.at[j]], ssems[b], add=True)

    def wait_scatter(j, b):
      pltpu.make_async_copy(rows[b], acc.at[dst_v.at[j]], ssems[b]).wait()

    def body(i, carry):
      j = NBUF * i
      for q in range(NBUF):
        wait_gather(q)
        start_scatter(j + q, q)
      for q in range(NBUF):
        wait_scatter(j + q, q)
        jn = j + NBUF + q

        @pl.when(jn < SEG)
        def _():
          start_gather(jn, q)
      return carry

    for seg in range(NCH // SEG):
      pltpu.sync_copy(eidx_hbm.at[0, wid, pl.ds(seg * SEG, SEG)], src_v)
      pltpu.sync_copy(eidx_hbm.at[1, wid, pl.ds(seg * SEG, SEG)], dst_v)
      for q in range(NBUF):
        start_gather(q, q)
      lax.fori_loop(0, SEG // NBUF, body, 0)
      for q in range(SEG % NBUF):
        j = (SEG // NBUF) * NBUF + q
        wait_gather(q)
        start_scatter(j, q)
      for q in range(SEG % NBUF):
        j = (SEG // NBUF) * NBUF + q
        wait_scatter(j, q)
    plsc.subcore_barrier()

    @pl.when(s < 15)
    def _():
      pltpu.sync_copy(acc.at[pl.ds(rbase, R_BIG)],
                      out_hbm.at[c, pl.ds(rbase, R_BIG)])

    @pl.when(s == 15)
    def _():
      pltpu.sync_copy(acc.at[pl.ds(rbase, R_LAST)],
                      out_hbm.at[c, pl.ds(rbase, R_LAST)])

  return agg


_agg128 = _make_agg(D_HID, 4, 25)
_agg64 = _make_agg(D_OUT, 10, NCH)


def _make_deg():
  mesh = plsc.VectorSubcoreMesh(core_axis_name="c", subcore_axis_name="s")

  @functools.partial(
      pl.kernel, mesh=mesh,
      out_type=jax.ShapeDtypeStruct((2, N_NODES, DEG_D), jnp.float32),
      compiler_params=pltpu.CompilerParams(use_tc_tiling_on_sc=False),
      scratch_types=[
          pltpu.VMEM((NCH, K), jnp.int32),
          pltpu.VMEM((K, DEG_D), jnp.float32),
          [pltpu.SemaphoreType.DMA for _ in range(NBUF_DEG)],
          pltpu.VMEM_SHARED((N_NODES, DEG_D), jnp.float32),
      ])
  def deg(eidx_hbm, ones_hbm, z_hbm, out_hbm, dst_v, ones_v, ssems, acc):
    c = lax.axis_index("c")
    s = lax.axis_index("s")
    wid = s * NC + c
    rbase = s * R_BIG

    @pl.when(s < 15)
    def _():
      pltpu.sync_copy(z_hbm.at[pl.ds(rbase, R_BIG)],
                      acc.at[pl.ds(rbase, R_BIG)])

    @pl.when(s == 15)
    def _():
      pltpu.sync_copy(z_hbm.at[pl.ds(rbase, R_LAST)],
                      acc.at[pl.ds(rbase, R_LAST)])

    pltpu.sync_copy(eidx_hbm.at[1, wid], dst_v)
    pltpu.sync_copy(ones_hbm, ones_v)
    plsc.subcore_barrier()

    def start_scatter(j, b):
      pltpu.async_copy(ones_v, acc.at[dst_v.at[j]], ssems[b], add=True)

    def wait_scatter(j, b):
      pltpu.make_async_copy(ones_v, acc.at[dst_v.at[j]], ssems[b]).wait()

    for q in range(NBUF_DEG):
      start_scatter(q, q)

    def body(i, carry):
      j = NBUF_DEG * i
      for q in range(NBUF_DEG):
        wait_scatter(j + q, q)
        jn = j + NBUF_DEG + q

        @pl.when(jn < NCH)
        def _():
          start_scatter(jn, q)
      return carry

    lax.fori_loop(0, NCH // NBUF_DEG, body, 0)
    for q in range(NCH % NBUF_DEG):
      wait_scatter((NCH // NBUF_DEG) * NBUF_DEG + q, q)
    plsc.subcore_barrier()

    @pl.when(s < 15)
    def _():
      pltpu.sync_copy(acc.at[pl.ds(rbase, R_BIG)],
                      out_hbm.at[c, pl.ds(rbase, R_BIG)])

    @pl.when(s == 15)
    def _():
      pltpu.sync_copy(acc.at[pl.ds(rbase, R_LAST)],
                      out_hbm.at[c, pl.ds(rbase, R_LAST)])

  return deg


_deg = _make_deg()


def _mm1_body(x_ref, w_ref, deg_ref, o_ref, dinv_ref):
  d = deg_ref[0, :, 0] + deg_ref[1, :, 0] + 1.0
  dinv = lax.rsqrt(d)[:, None]
  dinv_ref[...] = jnp.broadcast_to(dinv, dinv_ref.shape)
  h = jnp.dot(x_ref[...], w_ref[...], preferred_element_type=jnp.float32)
  o_ref[...] = h * dinv


def _mid_body(p_ref, h_ref, dinv_ref, b_ref, w_ref, o_ref):
  dinv = dinv_ref[:, 0][:, None]
  agg = p_ref[0] + p_ref[1] + h_ref[...]
  z = jnp.maximum(agg * dinv + b_ref[...], 0.0)
  o_ref[...] = jnp.dot(z * dinv, w_ref[...], preferred_element_type=jnp.float32)


def _fin_body(q_ref, h_ref, dinv_ref, b_ref, o_ref):
  dinv = dinv_ref[:, 0][:, None]
  o_ref[...] = (q_ref[0] + q_ref[1] + h_ref[...]) * dinv + b_ref[...]


def _mm1(x, W1, deg_raw):
  return pl.pallas_call(
      _mm1_body,
      grid=(N_NODES // BM,),
      in_specs=[
          pl.BlockSpec((BM, D_IN), lambda i: (i, 0)),
          pl.BlockSpec((D_IN, D_HID), lambda i: (0, 0)),
          pl.BlockSpec((2, BM, DEG_D), lambda i: (0, i, 0)),
      ],
      out_specs=[
          pl.BlockSpec((BM, D_HID), lambda i: (i, 0)),
          pl.BlockSpec((BM, 8), lambda i: (i, 0)),
      ],
      out_shape=[
          jax.ShapeDtypeStruct((N_NODES, D_HID), jnp.float32),
          jax.ShapeDtypeStruct((N_NODES, 8), jnp.float32),
      ],
  )(x, W1, deg_raw)


def _mid(p1, h1s, dinv, b1, W2):
  return pl.pallas_call(
      _mid_body,
      grid=(N_NODES // BM,),
      in_specs=[
          pl.BlockSpec((2, BM, D_HID), lambda i: (0, i, 0)),
          pl.BlockSpec((BM, D_HID), lambda i: (i, 0)),
          pl.BlockSpec((BM, 8), lambda i: (i, 0)),
          pl.BlockSpec((1, D_HID), lambda i: (0, 0)),
          pl.BlockSpec((D_HID, D_OUT), lambda i: (0, 0)),
      ],
      out_specs=pl.BlockSpec((BM, D_OUT), lambda i: (i, 0)),
      out_shape=jax.ShapeDtypeStruct((N_NODES, D_OUT), jnp.float32),
  )(p1, h1s, dinv, b1, W2)


def _fin(q, h2s, dinv, b2):
  return pl.pallas_call(
      _fin_body,
      grid=(N_NODES // BM,),
      in_specs=[
          pl.BlockSpec((2, BM, D_OUT), lambda i: (0, i, 0)),
          pl.BlockSpec((BM, D_OUT), lambda i: (i, 0)),
          pl.BlockSpec((BM, 8), lambda i: (i, 0)),
          pl.BlockSpec((1, D_OUT), lambda i: (0, 0)),
      ],
      out_specs=pl.BlockSpec((BM, D_OUT), lambda i: (i, 0)),
      out_shape=jax.ShapeDtypeStruct((N_NODES, D_OUT), jnp.float32),
  )(q, h2s, dinv, b2)


def kernel(x, edge_index, W1, b1, W2, b2):
  eidx = edge_index.reshape(2, NW, NCH, K)
  ones16 = jnp.ones((K, DEG_D), jnp.float32)
  z16 = jnp.zeros((N_NODES, DEG_D), jnp.float32)
  z128 = jnp.zeros((N_NODES, D_HID), jnp.float32)
  z64 = jnp.zeros((N_NODES, D_OUT), jnp.float32)

  deg_raw = _deg(eidx, ones16, z16)
  h1s, dinv = _mm1(x, W1, deg_raw)
  p1 = _agg128(h1s, eidx, z128)
  h2s = _mid(p1, h1s, dinv, b1.reshape(1, D_HID), W2)
  p2 = _agg64(h2s, eidx, z64)
  return _fin(p2, h2s, dinv, b2.reshape(1, D_OUT))


# speedup vs baseline: 43.1199x; 1.1892x over previous
import functools

import jax
import jax.numpy as jnp
from jax import lax
from jax.experimental import pallas as pl
from jax.experimental.pallas import tpu as pltpu
from jax.experimental.pallas import tpu_sc as plsc

N_NODES = 10000
N_EDGES = 320000
D_IN = 128
D_HID = 128
D_OUT = 64

NC = 2
NS = 16
NW = NC * NS
E_PER_W = N_EDGES // NW
K = 80
NCH = E_PER_W // K
R_BIG = 640
R_LAST = N_NODES - 15 * R_BIG
DEG_D = 16
NBUF_DEG = 8
BM = 1000


def _make_agg(D, NBUF, SEG):
  assert NCH % SEG == 0 and SEG >= NBUF
  mesh = plsc.VectorSubcoreMesh(core_axis_name="c", subcore_axis_name="s")

  @functools.partial(
      pl.kernel, mesh=mesh,
      out_type=jax.ShapeDtypeStruct((2, N_NODES, D), jnp.bfloat16),
      compiler_params=pltpu.CompilerParams(use_tc_tiling_on_sc=False),
      scratch_types=[
          pltpu.VMEM((SEG, K), jnp.int32),
          pltpu.VMEM((SEG, K), jnp.int32),
          [pltpu.VMEM((K, D), jnp.bfloat16) for _ in range(NBUF)],
          [pltpu.SemaphoreType.DMA for _ in range(NBUF)],
          [pltpu.SemaphoreType.DMA for _ in range(NBUF)],
          pltpu.VMEM_SHARED((N_NODES, D), jnp.bfloat16),
      ])
  def agg(h_hbm, eidx_hbm, z_hbm, out_hbm, src_v, dst_v, rows, gsems,
          ssems, acc):
    c = lax.axis_index("c")
    s = lax.axis_index("s")
    wid = s * NC + c
    rbase = s * R_BIG
    @pl.when(s < 15)
    def _():
      pltpu.sync_copy(z_hbm.at[pl.ds(rbase, R_BIG)],
                      acc.at[pl.ds(rbase, R_BIG)])

    @pl.when(s == 15)
    def _():
      pltpu.sync_copy(z_hbm.at[pl.ds(rbase, R_LAST)],
                      acc.at[pl.ds(rbase, R_LAST)])

    plsc.subcore_barrier()

    def start_gather(j, b):
      pltpu.async_copy(h_hbm.at[src_v.at[j]], rows[b], gsems[b])

    def wait_gather(b):
      pltpu.make_async_copy(h_hbm.at[src_v.at[0]], rows[b], gsems[b]).wait()

    def start_scatter(j, b):
      pltpu.async_copy(rows[b], acc.at[dst_v.at[j]], ssems[b], add=True)

    def wait_scatter(j, b):
      pltpu.make_async_copy(rows[b], acc.at[dst_v.at[j]], ssems[b]).wait()

    def body(i, carry):
      j = NBUF * i
      for q in range(NBUF):
        wait_gather(q)
        start_scatter(j + q, q)
      for q in range(NBUF):
        wait_scatter(j + q, q)
        jn = j + NBUF + q

        @pl.when(jn < SEG)
        def _():
          start_gather(jn, q)
      return carry

    for seg in range(NCH // SEG):
      pltpu.sync_copy(eidx_hbm.at[0, wid, pl.ds(seg * SEG, SEG)], src_v)
      pltpu.sync_copy(eidx_hbm.at[1, wid, pl.ds(seg * SEG, SEG)], dst_v)
      for q in range(NBUF):
        start_gather(q, q)
      lax.fori_loop(0, SEG // NBUF, body, 0)
      for q in range(SEG % NBUF):
        j = (SEG // NBUF) * NBUF + q
        wait_gather(q)
        start_scatter(j, q)
      for q in range(SEG % NBUF):
        j = (SEG // NBUF) * NBUF + q
        wait_scatter(j, q)
    plsc.subcore_barrier()

    @pl.when(s < 15)
    def _():
      pltpu.sync_copy(acc.at[pl.ds(rbase, R_BIG)],
                      out_hbm.at[c, pl.ds(rbase, R_BIG)])

    @pl.when(s == 15)
    def _():
      pltpu.sync_copy(acc.at[pl.ds(rbase, R_LAST)],
                      out_hbm.at[c, pl.ds(rbase, R_LAST)])

  return agg


_agg128 = _make_agg(D_HID, 6, 25)
_agg64 = _make_agg(D_OUT, 10, NCH)


def _make_deg():
  mesh = plsc.VectorSubcoreMesh(core_axis_name="c", subcore_axis_name="s")

  @functools.partial(
      pl.kernel, mesh=mesh,
      out_type=jax.ShapeDtypeStruct((2, N_NODES, DEG_D), jnp.float32),
      compiler_params=pltpu.CompilerParams(use_tc_tiling_on_sc=False),
      scratch_types=[
          pltpu.VMEM((NCH, K), jnp.int32),
          pltpu.VMEM((K, DEG_D), jnp.float32),
          [pltpu.SemaphoreType.DMA for _ in range(NBUF_DEG)],
          pltpu.VMEM_SHARED((N_NODES, DEG_D), jnp.float32),
      ])
  def deg(eidx_hbm, ones_hbm, z_hbm, out_hbm, dst_v, ones_v, ssems, acc):
    c = lax.axis_index("c")
    s = lax.axis_index("s")
    wid = s * NC + c
    rbase = s * R_BIG

    @pl.when(s < 15)
    def _():
      pltpu.sync_copy(z_hbm.at[pl.ds(rbase, R_BIG)],
                      acc.at[pl.ds(rbase, R_BIG)])

    @pl.when(s == 15)
    def _():
      pltpu.sync_copy(z_hbm.at[pl.ds(rbase, R_LAST)],
                      acc.at[pl.ds(rbase, R_LAST)])

    pltpu.sync_copy(eidx_hbm.at[1, wid], dst_v)
    pltpu.sync_copy(ones_hbm, ones_v)
    plsc.subcore_barrier()

    def start_scatter(j, b):
      pltpu.async_copy(ones_v, acc.at[dst_v.at[j]], ssems[b], add=True)

    def wait_scatter(j, b):
      pltpu.make_async_copy(ones_v, acc.at[dst_v.at[j]], ssems[b]).wait()

    for q in range(NBUF_DEG):
      start_scatter(q, q)

    def body(i, carry):
      j = NBUF_DEG * i
      for q in range(NBUF_DEG):
        wait_scatter(j + q, q)
        jn = j + NBUF_DEG + q

        @pl.when(jn < NCH)
        def _():
          start_scatter(jn, q)
      return carry

    lax.fori_loop(0, NCH // NBUF_DEG, body, 0)
    for q in range(NCH % NBUF_DEG):
      wait_scatter((NCH // NBUF_DEG) * NBUF_DEG + q, q)
    plsc.subcore_barrier()

    @pl.when(s < 15)
    def _():
      pltpu.sync_copy(acc.at[pl.ds(rbase, R_BIG)],
                      out_hbm.at[c, pl.ds(rbase, R_BIG)])

    @pl.when(s == 15)
    def _():
      pltpu.sync_copy(acc.at[pl.ds(rbase, R_LAST)],
                      out_hbm.at[c, pl.ds(rbase, R_LAST)])

  return deg


_deg = _make_deg()


def _mm1_body(x_ref, w_ref, deg_ref, o_ref, dinv_ref):
  d = deg_ref[0, :, 0] + deg_ref[1, :, 0] + 1.0
  dinv = lax.rsqrt(d)[:, None]
  dinv_ref[...] = jnp.broadcast_to(dinv, dinv_ref.shape)
  h = jnp.dot(x_ref[...], w_ref[...], preferred_element_type=jnp.float32)
  o_ref[...] = (h * dinv).astype(jnp.bfloat16)


def _mid_body(p_ref, h_ref, dinv_ref, b_ref, w_ref, o_ref):
  dinv = dinv_ref[:, 0][:, None]
  agg = (p_ref[0].astype(jnp.float32) + p_ref[1].astype(jnp.float32) +
         h_ref[...].astype(jnp.float32))
  z = jnp.maximum(agg * dinv + b_ref[...], 0.0)
  h2 = jnp.dot(z * dinv, w_ref[...], preferred_element_type=jnp.float32)
  o_ref[...] = h2.astype(jnp.bfloat16)


def _fin_body(q_ref, h_ref, dinv_ref, b_ref, o_ref):
  dinv = dinv_ref[:, 0][:, None]
  agg = (q_ref[0].astype(jnp.float32) + q_ref[1].astype(jnp.float32) +
         h_ref[...].astype(jnp.float32))
  o_ref[...] = agg * dinv + b_ref[...]


def _mm1(x, W1, deg_raw):
  return pl.pallas_call(
      _mm1_body,
      grid=(N_NODES // BM,),
      in_specs=[
          pl.BlockSpec((BM, D_IN), lambda i: (i, 0)),
          pl.BlockSpec((D_IN, D_HID), lambda i: (0, 0)),
          pl.BlockSpec((2, BM, DEG_D), lambda i: (0, i, 0)),
      ],
      out_specs=[
          pl.BlockSpec((BM, D_HID), lambda i: (i, 0)),
          pl.BlockSpec((BM, 8), lambda i: (i, 0)),
      ],
      out_shape=[
          jax.ShapeDtypeStruct((N_NODES, D_HID), jnp.bfloat16),
          jax.ShapeDtypeStruct((N_NODES, 8), jnp.float32),
      ],
  )(x, W1, deg_raw)


def _mid(p1, h1s, dinv, b1, W2):
  return pl.pallas_call(
      _mid_body,
      grid=(N_NODES // BM,),
      in_specs=[
          pl.BlockSpec((2, BM, D_HID), lambda i: (0, i, 0)),
          pl.BlockSpec((BM, D_HID), lambda i: (i, 0)),
          pl.BlockSpec((BM, 8), lambda i: (i, 0)),
          pl.BlockSpec((1, D_HID), lambda i: (0, 0)),
          pl.BlockSpec((D_HID, D_OUT), lambda i: (0, 0)),
      ],
      out_specs=pl.BlockSpec((BM, D_OUT), lambda i: (i, 0)),
      out_shape=jax.ShapeDtypeStruct((N_NODES, D_OUT), jnp.bfloat16),
  )(p1, h1s, dinv, b1, W2)


def _fin(q, h2s, dinv, b2):
  return pl.pallas_call(
      _fin_body,
      grid=(N_NODES // BM,),
      in_specs=[
          pl.BlockSpec((2, BM, D_OUT), lambda i: (0, i, 0)),
          pl.BlockSpec((BM, D_OUT), lambda i: (i, 0)),
          pl.BlockSpec((BM, 8), lambda i: (i, 0)),
          pl.BlockSpec((1, D_OUT), lambda i: (0, 0)),
      ],
      out_specs=pl.BlockSpec((BM, D_OUT), lambda i: (i, 0)),
      out_shape=jax.ShapeDtypeStruct((N_NODES, D_OUT), jnp.float32),
  )(q, h2s, dinv, b2)


def kernel(x, edge_index, W1, b1, W2, b2):
  eidx = edge_index.reshape(2, NW, NCH, K)
  ones16 = jnp.ones((K, DEG_D), jnp.float32)
  z16 = jnp.zeros((N_NODES, DEG_D), jnp.float32)
  z128 = jnp.zeros((N_NODES, D_HID), jnp.bfloat16)
  z64 = jnp.zeros((N_NODES, D_OUT), jnp.bfloat16)

  deg_raw = _deg(eidx, ones16, z16)
  h1s, dinv = _mm1(x, W1, deg_raw)
  p1 = _agg128(h1s, eidx, z128)
  h2s = _mid(p1, h1s, dinv, b1.reshape(1, D_HID), W2)
  p2 = _agg64(h2s, eidx, z64)
  return _fin(p2, h2s, dinv, b2.reshape(1, D_OUT))
